# baseline (speedup 1.0000x reference)
import functools

import jax
import jax.numpy as jnp
from jax import lax
from jax.experimental import pallas as pl
from jax.experimental.pallas import tpu as pltpu
from jax.experimental.pallas import tpu_sc as plsc

DIM = 16
LANES = 16


@functools.cache
def _build(batch, dim):
    info = plsc.get_sparse_core_info()
    nc, ns = info.num_cores, info.num_subcores
    nw = nc * ns
    assert batch % (8 * nw) == 0 and dim == LANES
    bpw = batch // nw
    groups = bpw // LANES

    mesh = plsc.VectorSubcoreMesh(core_axis_name="c", subcore_axis_name="s")

    @functools.partial(
        pl.kernel,
        mesh=mesh,
        compiler_params=pltpu.CompilerParams(
            needs_layout_passes=False, use_tc_tiling_on_sc=False),
        out_type=jax.ShapeDtypeStruct((batch,), jnp.float32),
        scratch_types=[
            pltpu.VMEM((bpw,), jnp.int32),
            pltpu.VMEM((bpw,), jnp.int32),
            pltpu.VMEM((bpw, dim), jnp.float32),
            pltpu.VMEM((bpw, dim), jnp.float32),
            pltpu.VMEM((bpw,), jnp.float32),
            pltpu.SemaphoreType.DMA,
        ],
    )
    def mf(user_hbm, item_hbm, utab_hbm, itab_hbm, out_hbm,
           uidx_v, iidx_v, urows_v, irows_v, out_v, sem):
        wid = lax.axis_index("s") * nc + lax.axis_index("c")
        base = wid * bpw
        pltpu.sync_copy(user_hbm.at[pl.ds(base, bpw)], uidx_v)
        pltpu.sync_copy(item_hbm.at[pl.ds(base, bpw)], iidx_v)
        cu = pltpu.async_copy(utab_hbm.at[uidx_v], urows_v, sem)
        ci = pltpu.async_copy(itab_hbm.at[iidx_v], irows_v, sem)
        cu.wait()
        ci.wait()

        lane = lax.iota(jnp.int32, LANES)

        def body(g, _):
            rows = g * LANES + lane
            acc = jnp.zeros((LANES,), jnp.float32)
            for k in range(dim):
                col = jnp.full((LANES,), k, jnp.int32)
                u = plsc.load_gather(urows_v, [rows, col])
                v = plsc.load_gather(irows_v, [rows, col])
                acc = acc + u * v
            out_v[pl.ds(g * LANES, LANES)] = acc
            return _

        lax.fori_loop(0, groups, body, None)
        pltpu.sync_copy(out_v, out_hbm.at[pl.ds(base, bpw)])

    return mf


def kernel(user, item, user_table, item_table):
    batch = user.shape[0]
    mf = _build(batch, user_table.shape[1])
    score = mf(user.astype(jnp.int32), item.astype(jnp.int32),
               user_table, item_table)
    return score[:, None]
